# Initial kernel scaffold; baseline (speedup 1.0000x reference)
import jax
import jax.numpy as jnp
from jax.experimental import pallas as pl


def kernel(features, edge_index, edge_weight, W, bias):
    raise NotImplementedError("write your pallas kernel here")


# speedup vs baseline: 2.8210x; 2.8210x over previous
import jax
import jax.numpy as jnp
from jax import lax
from jax.experimental import pallas as pl
from jax.experimental.pallas import tpu as pltpu
from jax.experimental.pallas import tpu_sc as plsc

N = 10000
E = 160000
D = 256
OUT = 256

NC = 2
NS = 16
L = 16

CG = 128
NROW = 10240
SLAB = NROW // NS
EPT = E // NS
C = 80
NCH = EPT // C
ZR = 128


def _spmm_body(xT, src_hbm, dst_hbm, w_hbm, y1T, s2T, Q, sbuf, dbuf,
               wbuf, rows, zbuf, sem):
  c = lax.axis_index("c")
  s = lax.axis_index("s")
  row0 = s * SLAB

  @pl.loop(0, ZR)
  def _(r):
    for j in range(CG // L):
      zbuf[r, pl.ds(j * L, L)] = jnp.zeros((L,), jnp.float32)

  def zero_slab():
    for t in range(SLAB // ZR):
      pltpu.sync_copy(zbuf, Q.at[pl.ds(row0 + t * ZR, ZR)])

  def spmm_pass(src_view):
    @pl.loop(0, NCH)
    def _(i):
      base = s * EPT + i * C
      pltpu.sync_copy(src_hbm.at[pl.ds(base, C)], sbuf)
      pltpu.sync_copy(dst_hbm.at[pl.ds(base, C)], dbuf)
      pltpu.sync_copy(w_hbm.at[pl.ds(base, C)], wbuf)
      pltpu.async_copy(src_view.at[sbuf], rows, sem).wait()

      @pl.loop(0, C // L)
      def _(k):
        w16 = wbuf[pl.ds(k * L, L)]
        for e in range(L):
          we = w16[e]
          r = k * L + e
          for j in range(CG // L):
            rows[r, pl.ds(j * L, L)] = rows[r, pl.ds(j * L, L)] * we

      pltpu.sync_copy(rows, Q.at[dbuf], add=True)

  zero_slab()
  plsc.subcore_barrier()

  spmm_pass(xT.at[c])
  plsc.subcore_barrier()

  pltpu.sync_copy(Q.at[pl.ds(row0, SLAB)], y1T.at[c, pl.ds(row0, SLAB)])
  zero_slab()
  plsc.subcore_barrier()

  spmm_pass(y1T.at[c])
  plsc.subcore_barrier()

  pltpu.sync_copy(Q.at[pl.ds(row0, SLAB)], s2T.at[c, pl.ds(row0, SLAB)])


def _make_spmm(interpret=False):
  return pl.kernel(
      _spmm_body,
      out_type=(
          jax.ShapeDtypeStruct((NC, NROW, CG), jnp.float32),
          jax.ShapeDtypeStruct((NC, NROW, CG), jnp.float32),
      ),
      mesh=plsc.VectorSubcoreMesh(
          core_axis_name="c", subcore_axis_name="s",
          num_cores=NC, num_subcores=NS),
      scratch_types=[
          pltpu.VMEM_SHARED((NROW, CG), jnp.float32),
          pltpu.VMEM((C,), jnp.int32),
          pltpu.VMEM((C,), jnp.int32),
          pltpu.VMEM((C,), jnp.float32),
          pltpu.VMEM((C, CG), jnp.float32),
          pltpu.VMEM((ZR, CG), jnp.float32),
          pltpu.SemaphoreType.DMA,
      ],
      interpret=interpret,
  )


_spmm = _make_spmm()


MMR = 400


def _mm_body(x0b, y1b, s2b, w0, w1, w2, bb, ob):
  a = jnp.dot(x0b[...], w0[...] - w2[...], preferred_element_type=jnp.float32)
  acc2 = None
  for q in range(NC):
    w1s = w1[q * CG:(q + 1) * CG, :]
    w2s = w2[q * CG:(q + 1) * CG, :]
    a = a + jnp.dot(y1b[q], w1s, preferred_element_type=jnp.float32)
    d2 = jnp.dot(s2b[q], w2s, preferred_element_type=jnp.float32)
    acc2 = d2 if acc2 is None else acc2 + d2
  ob[...] = a + 2.0 * acc2 + bb[...]


def _make_mm(interpret=False):
  return pl.pallas_call(
      _mm_body,
      grid=(N // MMR,),
      in_specs=[
          pl.BlockSpec((MMR, D), lambda i: (i, 0)),
          pl.BlockSpec((NC, MMR, CG), lambda i: (0, i, 0)),
          pl.BlockSpec((NC, MMR, CG), lambda i: (0, i, 0)),
          pl.BlockSpec((D, OUT), lambda i: (0, 0)),
          pl.BlockSpec((D, OUT), lambda i: (0, 0)),
          pl.BlockSpec((D, OUT), lambda i: (0, 0)),
          pl.BlockSpec((1, OUT), lambda i: (0, 0)),
      ],
      out_specs=pl.BlockSpec((MMR, OUT), lambda i: (i, 0)),
      out_shape=jax.ShapeDtypeStruct((N, OUT), jnp.float32),
      interpret=interpret,
  )


_mm = _make_mm()


@jax.jit
def kernel(features, edge_index, edge_weight, W, bias):
  x0 = features.reshape(N, D)
  xT = x0.reshape(N, NC, CG).transpose(1, 0, 2)
  xT = jnp.pad(xT, ((0, 0), (0, NROW - N), (0, 0)))
  y1T, s2T = _spmm(xT, edge_index[0], edge_index[1], edge_weight)

  W3 = W.reshape(D, 3, OUT)
  out = _mm(x0, y1T, s2T, W3[:, 0, :], W3[:, 1, :], W3[:, 2, :],
            bias.reshape(1, OUT))
  return out.reshape(1, N, OUT)


# speedup vs baseline: 6.3121x; 2.2375x over previous
import jax
import jax.numpy as jnp
from jax import lax
from jax.experimental import pallas as pl
from jax.experimental.pallas import tpu as pltpu
from jax.experimental.pallas import tpu_sc as plsc

N = 10000
E = 160000
D = 256
OUT = 256

NC = 2
NS = 16
L = 16

CG = 128
NROW = 10240
SLAB = NROW // NS
C = 128
NCH = 80
EPT = NCH * C
E_PAD = NS * EPT


def _spmm_body(xT, srcE, dstE, wE, y1T, s2T, Q,
               sb0, db0, wb0, rows0, sb1, db1, wb1, rows1,
               esem0, esem1, gsem0, gsem1):
  c = lax.axis_index("c")
  s = lax.axis_index("s")
  row0 = s * SLAB
  bufs = ((sb0, db0, wb0, rows0, esem0, gsem0),
          (sb1, db1, wb1, rows1, esem1, gsem1))

  def zero_slab():
    @pl.loop(0, C)
    def _(r):
      for j in range(CG // L):
        rows0[r, pl.ds(j * L, L)] = jnp.zeros((L,), jnp.float32)

    for t in range(SLAB // C):
      pltpu.sync_copy(rows0, Q.at[pl.ds(row0 + t * C, C)])

  def edges_issue(i, b):
    sb, db, wb, _, esem, _ = bufs[b]
    pltpu.async_copy(srcE.at[s, i], sb, esem)
    pltpu.async_copy(dstE.at[s, i], db, esem)
    pltpu.async_copy(wE.at[s, i], wb, esem)

  def edges_wait(i, b):
    sb, db, wb, _, esem, _ = bufs[b]
    pltpu.make_async_copy(srcE.at[s, i], sb, esem).wait()
    pltpu.make_async_copy(dstE.at[s, i], db, esem).wait()
    pltpu.make_async_copy(wE.at[s, i], wb, esem).wait()

  def scale(b):
    _, _, wb, rows, _, _ = bufs[b]

    @pl.loop(0, C // L)
    def _(k):
      w16 = wb[pl.ds(k * L, L)]
      for e in range(L):
        we = w16[e]
        r = k * L + e
        for j in range(CG // L):
          rows[r, pl.ds(j * L, L)] = rows[r, pl.ds(j * L, L)] * we

  def spmm_pass(src_view):
    def gather_issue(b):
      sb, _, _, rows, _, gsem = bufs[b]
      pltpu.async_copy(src_view.at[sb], rows, gsem)

    def gather_wait(b):
      sb, _, _, rows, _, gsem = bufs[b]
      pltpu.make_async_copy(src_view.at[sb], rows, gsem).wait()

    def step(i, b, next_gather, next_edges):
      _, db, _, rows, _, _ = bufs[b]
      gather_wait(b)
      if next_gather:
        edges_wait(i + 1, 1 - b)
        gather_issue(1 - b)
      scale(b)
      pltpu.sync_copy(rows, Q.at[db], add=True)
      if next_edges:
        edges_issue(i + 2, b)

    edges_issue(0, 0)
    edges_wait(0, 0)
    gather_issue(0)
    edges_issue(1, 1)

    @pl.loop(0, NCH // 2 - 1)
    def _(t):
      i0 = 2 * t
      step(i0, 0, True, True)
      step(i0 + 1, 1, True, True)

    step(NCH - 2, 0, True, False)
    step(NCH - 1, 1, False, False)

  zero_slab()
  plsc.subcore_barrier()

  spmm_pass(xT.at[c])
  plsc.subcore_barrier()

  pltpu.sync_copy(Q.at[pl.ds(row0, SLAB)], y1T.at[c, pl.ds(row0, SLAB)])
  zero_slab()
  plsc.subcore_barrier()

  spmm_pass(y1T.at[c])
  plsc.subcore_barrier()

  pltpu.sync_copy(Q.at[pl.ds(row0, SLAB)], s2T.at[c, pl.ds(row0, SLAB)])


def _make_spmm(interpret=False):
  return pl.kernel(
      _spmm_body,
      out_type=(
          jax.ShapeDtypeStruct((NC, NROW, CG), jnp.float32),
          jax.ShapeDtypeStruct((NC, NROW, CG), jnp.float32),
      ),
      mesh=plsc.VectorSubcoreMesh(
          core_axis_name="c", subcore_axis_name="s",
          num_cores=NC, num_subcores=NS),
      scratch_types=[
          pltpu.VMEM_SHARED((NROW, CG), jnp.float32),
          pltpu.VMEM((C,), jnp.int32),
          pltpu.VMEM((C,), jnp.int32),
          pltpu.VMEM((C,), jnp.float32),
          pltpu.VMEM((C, CG), jnp.float32),
          pltpu.VMEM((C,), jnp.int32),
          pltpu.VMEM((C,), jnp.int32),
          pltpu.VMEM((C,), jnp.float32),
          pltpu.VMEM((C, CG), jnp.float32),
          pltpu.SemaphoreType.DMA,
          pltpu.SemaphoreType.DMA,
          pltpu.SemaphoreType.DMA,
          pltpu.SemaphoreType.DMA,
      ],
      interpret=interpret,
  )


_spmm = _make_spmm()


MMR = 400


def _mm_body(x0b, y1b, s2b, w0, w1, w2, bb, ob):
  a = jnp.dot(x0b[...], w0[...] - w2[...], preferred_element_type=jnp.float32)
  acc2 = None
  for q in range(NC):
    w1s = w1[q * CG:(q + 1) * CG, :]
    w2s = w2[q * CG:(q + 1) * CG, :]
    a = a + jnp.dot(y1b[q], w1s, preferred_element_type=jnp.float32)
    d2 = jnp.dot(s2b[q], w2s, preferred_element_type=jnp.float32)
    acc2 = d2 if acc2 is None else acc2 + d2
  ob[...] = a + 2.0 * acc2 + bb[...]


def _make_mm(interpret=False):
  return pl.pallas_call(
      _mm_body,
      grid=(N // MMR,),
      in_specs=[
          pl.BlockSpec((MMR, D), lambda i: (i, 0)),
          pl.BlockSpec((NC, MMR, CG), lambda i: (0, i, 0)),
          pl.BlockSpec((NC, MMR, CG), lambda i: (0, i, 0)),
          pl.BlockSpec((D, OUT), lambda i: (0, 0)),
          pl.BlockSpec((D, OUT), lambda i: (0, 0)),
          pl.BlockSpec((D, OUT), lambda i: (0, 0)),
          pl.BlockSpec((1, OUT), lambda i: (0, 0)),
      ],
      out_specs=pl.BlockSpec((MMR, OUT), lambda i: (i, 0)),
      out_shape=jax.ShapeDtypeStruct((N, OUT), jnp.float32),
      interpret=interpret,
  )


_mm = _make_mm()


@jax.jit
def kernel(features, edge_index, edge_weight, W, bias):
  x0 = features.reshape(N, D)
  xT = x0.reshape(N, NC, CG).transpose(1, 0, 2)
  xT = jnp.pad(xT, ((0, 0), (0, NROW - N), (0, 0)))

  pad = E_PAD - E
  iot = jnp.arange(pad, dtype=jnp.int32)
  srcp = jnp.concatenate([edge_index[0], iot % N]).reshape(NS, NCH, C)
  dstp = jnp.concatenate([edge_index[1], N + iot % (NROW - N)]).reshape(
      NS, NCH, C)
  wp = jnp.concatenate(
      [edge_weight, jnp.zeros((pad,), jnp.float32)]).reshape(NS, NCH, C)
  y1T, s2T = _spmm(xT, srcp, dstp, wp)

  W3 = W.reshape(D, 3, OUT)
  out = _mm(x0, y1T, s2T, W3[:, 0, :], W3[:, 1, :], W3[:, 2, :],
            bias.reshape(1, OUT))
  return out.reshape(1, N, OUT)
